# baseline (speedup 1.0000x reference)
import jax
import jax.numpy as jnp
from jax.experimental import pallas as pl
from jax.experimental.pallas import tpu as pltpu

_NCHUNK = 8
_N = 1024


def _fused_body(adj_hbm, feats_ref, w1_ref, b1_ref, w2_ref, b2_ref,
                tot_ref, wq_ref, bq_ref, wk_ref, bk_ref, wv_ref, bv_ref,
                ws_ref, bs_ref, speak_ref, tot_out_ref, a_scr, af_scr, sems):
    f32 = jnp.float32
    C = _N // _NCHUNK
    copies = [
        pltpu.make_async_copy(
            adj_hbm.at[pl.ds(i * C, C), :],
            a_scr.at[pl.ds(i * C, C), :],
            sems.at[i],
        )
        for i in range(_NCHUNK)
    ]
    for c in copies:
        c.start()

    h1 = jnp.dot(feats_ref[...], w1_ref[...], preferred_element_type=f32)

    colsum = jnp.zeros((1, _N), f32)
    for i, c in enumerate(copies):
        c.wait()
        chunk = a_scr[pl.ds(i * C, C), :].astype(f32)
        af_scr[pl.ds(i * C, C), :] = chunk
        colsum = colsum + jnp.sum(chunk, axis=0, keepdims=True)

    deg = colsum + 1.0
    dis = jax.lax.rsqrt(deg)
    discol = dis.reshape(_N, 1)

    a = af_scr[...]
    hp1 = discol * h1
    t1 = jax.lax.dot_general(a, hp1, (((0,), (0,)), ((), ())),
                             preferred_element_type=f32) + hp1
    g1 = jnp.maximum(discol * t1 + b1_ref[...], 0.0)

    h2 = jnp.dot(g1, w2_ref[...], preferred_element_type=f32)
    hp2 = discol * h2
    t2 = jax.lax.dot_general(a, hp2, (((0,), (0,)), ((), ())),
                             preferred_element_type=f32) + hp2
    g2 = discol * t2 + b2_ref[...]

    q = jnp.dot(g2, wq_ref[...], preferred_element_type=f32) + bq_ref[...]
    k = jnp.dot(tot_ref[...], wk_ref[...], preferred_element_type=f32) + bk_ref[...]
    logits = jnp.sum(q * k, axis=1, keepdims=True) * (1.0 / 8.0)
    rel = jax.nn.sigmoid(logits)
    rg = jax.lax.dot_general(rel, g2, (((0,), (0,)), ((), ())),
                             preferred_element_type=f32)
    summed = jnp.dot(rg, wv_ref[...], preferred_element_type=f32) \
        + jnp.sum(rel) * bv_ref[...]
    new_tot = tot_ref[...] + summed
    tot_out_ref[...] = new_tot
    speak_ref[...] = jnp.dot(new_tot, ws_ref[...],
                             preferred_element_type=f32) + bs_ref[...]


def kernel(big_batch_positions, big_batched_adjacency_pruned, ego_mask_batch,
           timestep, W1, b1, W2, b2, trainOT, Wq, bq, Wk, bk, Wv, bv,
           Wspeak, bspeak):
    feats = big_batch_positions[timestep]
    adj = big_batched_adjacency_pruned[timestep]
    f32 = jnp.float32

    vmem = pl.BlockSpec(memory_space=pltpu.VMEM)
    speak, new_tot = pl.pallas_call(
        _fused_body,
        in_specs=[pl.BlockSpec(memory_space=pl.ANY)] + [vmem] * 14,
        out_shape=(
            jax.ShapeDtypeStruct((1, Wspeak.shape[1]), f32),
            jax.ShapeDtypeStruct((1, trainOT.shape[0]), f32),
        ),
        scratch_shapes=[
            pltpu.VMEM((_N, _N), jnp.int32),
            pltpu.VMEM((_N, _N), f32),
            pltpu.SemaphoreType.DMA((_NCHUNK,)),
        ],
    )(
        adj, feats,
        W1, b1.reshape(1, -1), W2, b2.reshape(1, -1),
        trainOT.reshape(1, -1),
        Wq, bq.reshape(1, -1), Wk, bk.reshape(1, -1),
        Wv, bv.reshape(1, -1), Wspeak, bspeak.reshape(1, -1),
    )
    out = speak.reshape(1, -1, 4)
    return (out, new_tot.reshape(-1))


# speedup vs baseline: 1.0310x; 1.0310x over previous
import jax
import jax.numpy as jnp
from jax.experimental import pallas as pl
from jax.experimental.pallas import tpu as pltpu

_NCHUNK = 8
_N = 1024


def _fused_body(adj_hbm, ws_hbm, feats_ref, w1_ref, b1_ref, w2_ref, b2_ref,
                tot_ref, wq_ref, bq_ref, wk_ref, bk_ref, wv_ref, bv_ref,
                bs_ref, speak_ref, tot_out_ref, a_scr, ab_scr, ws_scr, sems,
                ws_sem):
    f32 = jnp.float32
    bf16 = jnp.bfloat16
    C = _N // _NCHUNK
    copies = [
        pltpu.make_async_copy(
            adj_hbm.at[pl.ds(i * C, C), :],
            a_scr.at[pl.ds(i * C, C), :],
            sems.at[i],
        )
        for i in range(_NCHUNK)
    ]
    for c in copies:
        c.start()
    ws_copy = pltpu.make_async_copy(ws_hbm, ws_scr, ws_sem)
    ws_copy.start()

    h1 = jnp.dot(feats_ref[...], w1_ref[...], preferred_element_type=f32)

    colsum = jnp.zeros((1, _N), f32)
    for i, c in enumerate(copies):
        c.wait()
        chunk = a_scr[pl.ds(i * C, C), :]
        ab_scr[pl.ds(i * C, C), :] = chunk.astype(bf16)
        colsum = colsum + jnp.sum(chunk.astype(f32), axis=0, keepdims=True)

    deg = colsum + 1.0
    dis = jax.lax.rsqrt(deg)
    discol = dis.reshape(_N, 1)
    a = ab_scr[...]

    def agg(hp):
        hi = hp.astype(bf16)
        lo = (hp - hi.astype(f32)).astype(bf16)
        both = jnp.concatenate([hi, lo], axis=1)
        r = jax.lax.dot_general(a, both, (((0,), (0,)), ((), ())),
                                preferred_element_type=f32)
        return r[:, :hp.shape[1]] + r[:, hp.shape[1]:]

    hp1 = discol * h1
    g1 = jnp.maximum(discol * (agg(hp1) + hp1) + b1_ref[...], 0.0)

    h2 = jnp.dot(g1, w2_ref[...], preferred_element_type=f32)
    hp2 = discol * h2
    g2 = discol * (agg(hp2) + hp2) + b2_ref[...]

    q = jnp.dot(g2, wq_ref[...], preferred_element_type=f32) + bq_ref[...]
    k = jnp.dot(tot_ref[...], wk_ref[...], preferred_element_type=f32) + bk_ref[...]
    logits = jnp.sum(q * k, axis=1, keepdims=True) * (1.0 / 8.0)
    rel = jax.nn.sigmoid(logits)
    rg = jax.lax.dot_general(rel, g2, (((0,), (0,)), ((), ())),
                             preferred_element_type=f32)
    summed = jnp.dot(rg, wv_ref[...], preferred_element_type=f32) \
        + jnp.sum(rel) * bv_ref[...]
    new_tot = tot_ref[...] + summed
    tot_out_ref[...] = new_tot
    ws_copy.wait()
    speak_ref[...] = jnp.dot(new_tot, ws_scr[...],
                             preferred_element_type=f32) + bs_ref[...]


def kernel(big_batch_positions, big_batched_adjacency_pruned, ego_mask_batch,
           timestep, W1, b1, W2, b2, trainOT, Wq, bq, Wk, bk, Wv, bv,
           Wspeak, bspeak):
    feats = big_batch_positions[timestep]
    adj = big_batched_adjacency_pruned[timestep]
    f32 = jnp.float32

    vmem = pl.BlockSpec(memory_space=pltpu.VMEM)
    hbm = pl.BlockSpec(memory_space=pl.ANY)
    speak, new_tot = pl.pallas_call(
        _fused_body,
        in_specs=[hbm, hbm] + [vmem] * 13,
        out_shape=(
            jax.ShapeDtypeStruct((1, Wspeak.shape[1]), f32),
            jax.ShapeDtypeStruct((1, trainOT.shape[0]), f32),
        ),
        scratch_shapes=[
            pltpu.VMEM((_N, _N), jnp.int32),
            pltpu.VMEM((_N, _N), jnp.bfloat16),
            pltpu.VMEM(Wspeak.shape, f32),
            pltpu.SemaphoreType.DMA((_NCHUNK,)),
            pltpu.SemaphoreType.DMA,
        ],
    )(
        adj, Wspeak, feats,
        W1, b1.reshape(1, -1), W2, b2.reshape(1, -1),
        trainOT.reshape(1, -1),
        Wq, bq.reshape(1, -1), Wk, bk.reshape(1, -1),
        Wv, bv.reshape(1, -1), bspeak.reshape(1, -1),
    )
    out = speak.reshape(1, -1, 4)
    return (out, new_tot.reshape(-1))


# speedup vs baseline: 1.0393x; 1.0081x over previous
import jax
import jax.numpy as jnp
from jax.experimental import pallas as pl
from jax.experimental.pallas import tpu as pltpu

_N = 1024


def _fused_body(adj_ref, feats_ref, w1_ref, b1_ref, w2_ref, b2_ref,
                tot_ref, wq_ref, bq_ref, wk_ref, bk_ref, wv_ref, bv_ref,
                ws_ref, bs_ref, speak_ref, tot_out_ref):
    f32 = jnp.float32
    bf16 = jnp.bfloat16
    a = adj_ref[...].astype(bf16)
    ones_row = jnp.ones((1, _N), bf16)
    colsum = jax.lax.dot_general(ones_row, a, (((1,), (0,)), ((), ())),
                                 preferred_element_type=f32)
    deg = colsum + 1.0
    dis = jax.lax.rsqrt(deg)
    discol = dis.reshape(_N, 1)

    def agg(hp):
        hi = hp.astype(bf16)
        lo = (hp - hi.astype(f32)).astype(bf16)
        both = jnp.concatenate([hi, lo], axis=1)
        r = jax.lax.dot_general(a, both, (((0,), (0,)), ((), ())),
                                preferred_element_type=f32)
        return r[:, :hp.shape[1]] + r[:, hp.shape[1]:]

    h1 = jnp.dot(feats_ref[...], w1_ref[...], preferred_element_type=f32)
    hp1 = discol * h1
    g1 = jnp.maximum(discol * (agg(hp1) + hp1) + b1_ref[...], 0.0)

    h2 = jnp.dot(g1, w2_ref[...], preferred_element_type=f32)
    hp2 = discol * h2
    g2 = discol * (agg(hp2) + hp2) + b2_ref[...]

    q = jnp.dot(g2, wq_ref[...], preferred_element_type=f32) + bq_ref[...]
    k = jnp.dot(tot_ref[...], wk_ref[...], preferred_element_type=f32) + bk_ref[...]
    logits = jnp.sum(q * k, axis=1, keepdims=True) * (1.0 / 8.0)
    rel = jax.nn.sigmoid(logits)
    rg = jax.lax.dot_general(rel, g2, (((0,), (0,)), ((), ())),
                             preferred_element_type=f32)
    summed = jnp.dot(rg, wv_ref[...], preferred_element_type=f32) \
        + jnp.sum(rel) * bv_ref[...]
    new_tot = tot_ref[...] + summed
    tot_out_ref[...] = new_tot
    speak_ref[...] = jnp.dot(new_tot, ws_ref[...],
                             preferred_element_type=f32) + bs_ref[...]


def kernel(big_batch_positions, big_batched_adjacency_pruned, ego_mask_batch,
           timestep, W1, b1, W2, b2, trainOT, Wq, bq, Wk, bk, Wv, bv,
           Wspeak, bspeak):
    feats = big_batch_positions[timestep]
    adj = big_batched_adjacency_pruned[timestep]
    f32 = jnp.float32

    speak, new_tot = pl.pallas_call(
        _fused_body,
        out_shape=(
            jax.ShapeDtypeStruct((1, Wspeak.shape[1]), f32),
            jax.ShapeDtypeStruct((1, trainOT.shape[0]), f32),
        ),
    )(
        adj, feats,
        W1, b1.reshape(1, -1), W2, b2.reshape(1, -1),
        trainOT.reshape(1, -1),
        Wq, bq.reshape(1, -1), Wk, bk.reshape(1, -1),
        Wv, bv.reshape(1, -1), Wspeak, bspeak.reshape(1, -1),
    )
    out = speak.reshape(1, -1, 4)
    return (out, new_tot.reshape(-1))


# speedup vs baseline: 1.0876x; 1.0464x over previous
import jax
import jax.numpy as jnp
from jax.experimental import pallas as pl
from jax.experimental.pallas import tpu as pltpu


def _fused_body(ws_hbm, feats_ref, adj_ref, w1_ref, b1_ref, w2_ref, b2_ref,
                tot_ref, wq_ref, bq_ref, wk_ref, bk_ref, wv_ref, bv_ref,
                bs_ref, speak_ref, tot_out_ref, ws_scr, ws_sem):
    f32 = jnp.float32
    ws_copy = pltpu.make_async_copy(ws_hbm, ws_scr, ws_sem)
    ws_copy.start()

    a = adj_ref[...].astype(f32)
    deg = jnp.sum(a, axis=0) + 1.0
    dis = jax.lax.rsqrt(deg)
    discol = dis[:, None]

    h1 = jnp.dot(feats_ref[...], w1_ref[...], preferred_element_type=f32)
    hp1 = discol * h1
    t1 = jax.lax.dot_general(a, hp1, (((0,), (0,)), ((), ())),
                             preferred_element_type=f32) + hp1
    g1 = jnp.maximum(discol * t1 + b1_ref[...], 0.0)

    h2 = jnp.dot(g1, w2_ref[...], preferred_element_type=f32)
    hp2 = discol * h2
    t2 = jax.lax.dot_general(a, hp2, (((0,), (0,)), ((), ())),
                             preferred_element_type=f32) + hp2
    g2 = discol * t2 + b2_ref[...]

    q = jnp.dot(g2, wq_ref[...], preferred_element_type=f32) + bq_ref[...]
    k = jnp.dot(tot_ref[...], wk_ref[...], preferred_element_type=f32) + bk_ref[...]
    logits = jnp.sum(q * k, axis=1, keepdims=True) * (1.0 / 8.0)
    rel = jax.nn.sigmoid(logits)
    rg = jax.lax.dot_general(rel, g2, (((0,), (0,)), ((), ())),
                             preferred_element_type=f32)
    summed = jnp.dot(rg, wv_ref[...], preferred_element_type=f32) \
        + jnp.sum(rel) * bv_ref[...]
    new_tot = tot_ref[...] + summed
    tot_out_ref[...] = new_tot
    ws_copy.wait()
    speak_ref[...] = jnp.dot(new_tot, ws_scr[...],
                             preferred_element_type=f32) + bs_ref[...]


def kernel(big_batch_positions, big_batched_adjacency_pruned, ego_mask_batch,
           timestep, W1, b1, W2, b2, trainOT, Wq, bq, Wk, bk, Wv, bv,
           Wspeak, bspeak):
    feats = big_batch_positions[timestep]
    adj = big_batched_adjacency_pruned[timestep]
    f32 = jnp.float32

    vmem = pl.BlockSpec(memory_space=pltpu.VMEM)
    speak, new_tot = pl.pallas_call(
        _fused_body,
        in_specs=[pl.BlockSpec(memory_space=pl.ANY)] + [vmem] * 14,
        out_shape=(
            jax.ShapeDtypeStruct((1, Wspeak.shape[1]), f32),
            jax.ShapeDtypeStruct((1, trainOT.shape[0]), f32),
        ),
        scratch_shapes=[
            pltpu.VMEM(Wspeak.shape, f32),
            pltpu.SemaphoreType.DMA,
        ],
    )(
        Wspeak, feats, adj,
        W1, b1.reshape(1, -1), W2, b2.reshape(1, -1),
        trainOT.reshape(1, -1),
        Wq, bq.reshape(1, -1), Wk, bk.reshape(1, -1),
        Wv, bv.reshape(1, -1), bspeak.reshape(1, -1),
    )
    out = speak.reshape(1, -1, 4)
    return (out, new_tot.reshape(-1))


# speedup vs baseline: 1.1718x; 1.0774x over previous
import jax
import jax.numpy as jnp
from jax.experimental import pallas as pl
from jax.experimental.pallas import tpu as pltpu

_N = 1024
_H = _N // 2


def _fused_body(ws_hbm, feats_ref, adj_ref, w1_ref, b1_ref, w2_ref, b2_ref,
                tot_ref, wq_ref, bq_ref, wk_ref, bk_ref, wv_ref, bv_ref,
                bs_ref, speak_ref, tot_out_ref, ws_scr, ws_sem):
    f32 = jnp.float32
    ws_copy = pltpu.make_async_copy(ws_hbm, ws_scr, ws_sem)
    ws_copy.start()

    ai = adj_ref[...]
    deg = jnp.sum(ai, axis=0).astype(f32) + 1.0
    dis = jax.lax.rsqrt(deg)
    discol = dis[:, None]
    a0 = ai[:_H, :].astype(f32)
    a1 = ai[_H:, :].astype(f32)

    def agg(hp):
        p0 = jax.lax.dot_general(a0, hp[:_H, :], (((0,), (0,)), ((), ())),
                                 preferred_element_type=f32)
        p1 = jax.lax.dot_general(a1, hp[_H:, :], (((0,), (0,)), ((), ())),
                                 preferred_element_type=f32)
        return p0 + p1 + hp

    h1 = jnp.dot(feats_ref[...], w1_ref[...], preferred_element_type=f32)
    hp1 = discol * h1
    g1 = jnp.maximum(discol * agg(hp1) + b1_ref[...], 0.0)

    h2 = jnp.dot(g1, w2_ref[...], preferred_element_type=f32)
    hp2 = discol * h2
    g2 = discol * agg(hp2) + b2_ref[...]

    q = jnp.dot(g2, wq_ref[...], preferred_element_type=f32) + bq_ref[...]
    k = jnp.dot(tot_ref[...], wk_ref[...], preferred_element_type=f32) + bk_ref[...]
    logits = jnp.sum(q * k, axis=1, keepdims=True) * (1.0 / 8.0)
    rel = jax.nn.sigmoid(logits)
    rg = jax.lax.dot_general(rel, g2, (((0,), (0,)), ((), ())),
                             preferred_element_type=f32)
    summed = jnp.dot(rg, wv_ref[...], preferred_element_type=f32) \
        + jnp.sum(rel) * bv_ref[...]
    new_tot = tot_ref[...] + summed
    tot_out_ref[...] = new_tot
    ws_copy.wait()
    speak_ref[...] = jnp.dot(new_tot, ws_scr[...],
                             preferred_element_type=f32) + bs_ref[...]


def kernel(big_batch_positions, big_batched_adjacency_pruned, ego_mask_batch,
           timestep, W1, b1, W2, b2, trainOT, Wq, bq, Wk, bk, Wv, bv,
           Wspeak, bspeak):
    feats = big_batch_positions[timestep]
    adj = big_batched_adjacency_pruned[timestep]
    f32 = jnp.float32

    vmem = pl.BlockSpec(memory_space=pltpu.VMEM)
    speak, new_tot = pl.pallas_call(
        _fused_body,
        in_specs=[pl.BlockSpec(memory_space=pl.ANY)] + [vmem] * 14,
        out_shape=(
            jax.ShapeDtypeStruct((1, Wspeak.shape[1]), f32),
            jax.ShapeDtypeStruct((1, trainOT.shape[0]), f32),
        ),
        scratch_shapes=[
            pltpu.VMEM(Wspeak.shape, f32),
            pltpu.SemaphoreType.DMA,
        ],
    )(
        Wspeak, feats, adj,
        W1, b1.reshape(1, -1), W2, b2.reshape(1, -1),
        trainOT.reshape(1, -1),
        Wq, bq.reshape(1, -1), Wk, bk.reshape(1, -1),
        Wv, bv.reshape(1, -1), bspeak.reshape(1, -1),
    )
    out = speak.reshape(1, -1, 4)
    return (out, new_tot.reshape(-1))
